# baseline (speedup 1.0000x reference)
import functools

import jax
import jax.numpy as jnp
from jax.experimental import pallas as pl
from jax.experimental.pallas import tpu as pltpu

_NB = 8
_NB_STATS = 8
_K = 3
_DIL = 2
_PAD = (_K - 1) * _DIL // 2


def _conv_rows(xb, w):
    nb, l, c = xb.shape
    z = jnp.zeros((nb, _PAD, c), xb.dtype)
    xm = jnp.concatenate([z, xb[:, : l - _PAD, :]], axis=1)
    xp = jnp.concatenate([xb[:, _PAD:, :], z], axis=1)
    a = jnp.concatenate([xm, xb, xp], axis=2).reshape(nb * l, _K * c)
    return jnp.dot(a, w, preferred_element_type=jnp.float32)


def _stats_kernel(x_ref, w_ref, stats_ref, x16_ref):
    xb = x_ref[...].astype(jnp.bfloat16)
    x16_ref[...] = xb
    conv = _conv_rows(xb, w_ref[...])
    s = jnp.sum(conv, axis=0, keepdims=True)
    s2 = jnp.sum(conv * conv, axis=0, keepdims=True)
    stats_ref[0, 0:1, :] = s
    stats_ref[0, 1:2, :] = s2


def _apply_kernel(x16_ref, w_ref, stats_ref, g_ref, bt_ref, o_ref, *, count):
    mean_raw = jnp.sum(stats_ref[:, 0, :], axis=0, keepdims=True) / count
    ex2 = jnp.sum(stats_ref[:, 1, :], axis=0, keepdims=True) / count
    var = ex2 - mean_raw * mean_raw
    scale = g_ref[...] * jax.lax.rsqrt(var + 1e-5)
    shift = bt_ref[...] - mean_raw * scale
    w2 = (w_ref[...].astype(jnp.float32) * scale).astype(jnp.bfloat16)
    y = _conv_rows(x16_ref[...], w2) + shift
    nb, l, _ = x16_ref.shape
    o_ref[...] = y.reshape(nb, l, y.shape[1])


@jax.jit
def _forward(x, conv_weight, conv_bias, bn_gamma, bn_beta):
    B, L, C_in = x.shape
    C_out = conv_weight.shape[0]
    L_out = L

    w_flat = jnp.transpose(conv_weight, (2, 1, 0)).reshape(_K * C_in, C_out)
    w16 = w_flat.astype(jnp.bfloat16)

    nblk = B // _NB
    nblk_s = B // _NB_STATS
    cparams = pltpu.CompilerParams(
        dimension_semantics=("parallel",),
        vmem_limit_bytes=60 * 1024 * 1024)

    stats, x16 = pl.pallas_call(
        _stats_kernel,
        out_shape=(
            jax.ShapeDtypeStruct((nblk_s, 2, C_out), jnp.float32),
            jax.ShapeDtypeStruct((B, L, C_in), jnp.bfloat16),
        ),
        grid=(nblk_s,),
        in_specs=[
            pl.BlockSpec((_NB_STATS, L, C_in), lambda b: (b, 0, 0)),
            pl.BlockSpec((_K * C_in, C_out), lambda b: (0, 0)),
        ],
        out_specs=(
            pl.BlockSpec((1, 2, C_out), lambda b: (b, 0, 0)),
            pl.BlockSpec((_NB_STATS, L, C_in), lambda b: (b, 0, 0)),
        ),
        compiler_params=cparams,
    )(x, w16)

    g_row = bn_gamma.reshape(1, C_out)
    bt_row = bn_beta.reshape(1, C_out)
    y = pl.pallas_call(
        functools.partial(_apply_kernel, count=B * L_out),
        out_shape=jax.ShapeDtypeStruct((B, L_out, C_out), jnp.float32),
        grid=(nblk,),
        in_specs=[
            pl.BlockSpec((_NB, L, C_in), lambda b: (b, 0, 0)),
            pl.BlockSpec((_K * C_in, C_out), lambda b: (0, 0)),
            pl.BlockSpec((nblk_s, 2, C_out), lambda b: (0, 0, 0)),
            pl.BlockSpec((1, C_out), lambda b: (0, 0)),
            pl.BlockSpec((1, C_out), lambda b: (0, 0)),
        ],
        out_specs=pl.BlockSpec((_NB, L_out, C_out), lambda b: (b, 0, 0)),
        compiler_params=cparams,
    )(x16, w16, stats, g_row, bt_row)

    return y


def kernel(x, conv_weight, conv_bias, bn_gamma, bn_beta):
    return _forward(x, conv_weight, conv_bias, bn_gamma, bn_beta)


# speedup vs baseline: 1.0066x; 1.0066x over previous
import functools

import jax
import jax.numpy as jnp
from jax.experimental import pallas as pl
from jax.experimental.pallas import tpu as pltpu

_NB = 8
_NB_STATS = 16
_K = 3
_DIL = 2
_PAD = (_K - 1) * _DIL // 2


def _conv_rows(xb, w):
    nb, l, c = xb.shape
    z = jnp.zeros((nb, _PAD, c), xb.dtype)
    xm = jnp.concatenate([z, xb[:, : l - _PAD, :]], axis=1)
    xp = jnp.concatenate([xb[:, _PAD:, :], z], axis=1)
    a = jnp.concatenate([xm, xb, xp], axis=2).reshape(nb * l, _K * c)
    return jnp.dot(a, w, preferred_element_type=jnp.float32)


def _stats_kernel(x_ref, w_ref, stats_ref, x16_ref):
    xb = x_ref[...].astype(jnp.bfloat16)
    x16_ref[...] = xb
    conv = _conv_rows(xb, w_ref[...])
    s = jnp.sum(conv, axis=0, keepdims=True)
    s2 = jnp.sum(conv * conv, axis=0, keepdims=True)
    stats_ref[0, 0:1, :] = s
    stats_ref[0, 1:2, :] = s2


def _apply_kernel(x16_ref, w_ref, stats_ref, g_ref, bt_ref, o_ref, *, count):
    mean_raw = jnp.sum(stats_ref[:, 0, :], axis=0, keepdims=True) / count
    ex2 = jnp.sum(stats_ref[:, 1, :], axis=0, keepdims=True) / count
    var = ex2 - mean_raw * mean_raw
    scale = g_ref[...] * jax.lax.rsqrt(var + 1e-5)
    shift = bt_ref[...] - mean_raw * scale
    y = _conv_rows(x16_ref[...], w_ref[...]) * scale + shift
    nb, l, _ = x16_ref.shape
    o_ref[...] = y.reshape(nb, l, y.shape[1])


@jax.jit
def _forward(x, conv_weight, conv_bias, bn_gamma, bn_beta):
    B, L, C_in = x.shape
    C_out = conv_weight.shape[0]
    L_out = L

    w_flat = jnp.transpose(conv_weight, (2, 1, 0)).reshape(_K * C_in, C_out)
    w16 = w_flat.astype(jnp.bfloat16)

    nblk = B // _NB
    nblk_s = B // _NB_STATS
    cparams = pltpu.CompilerParams(
        dimension_semantics=("parallel",),
        vmem_limit_bytes=60 * 1024 * 1024)

    stats, x16 = pl.pallas_call(
        _stats_kernel,
        out_shape=(
            jax.ShapeDtypeStruct((nblk_s, 2, C_out), jnp.float32),
            jax.ShapeDtypeStruct((B, L, C_in), jnp.bfloat16),
        ),
        grid=(nblk_s,),
        in_specs=[
            pl.BlockSpec((_NB_STATS, L, C_in), lambda b: (b, 0, 0)),
            pl.BlockSpec((_K * C_in, C_out), lambda b: (0, 0)),
        ],
        out_specs=(
            pl.BlockSpec((1, 2, C_out), lambda b: (b, 0, 0)),
            pl.BlockSpec((_NB_STATS, L, C_in), lambda b: (b, 0, 0)),
        ),
        compiler_params=cparams,
    )(x, w16)

    g_row = bn_gamma.reshape(1, C_out)
    bt_row = bn_beta.reshape(1, C_out)
    y = pl.pallas_call(
        functools.partial(_apply_kernel, count=B * L_out),
        out_shape=jax.ShapeDtypeStruct((B, L_out, C_out), jnp.float32),
        grid=(nblk,),
        in_specs=[
            pl.BlockSpec((_NB, L, C_in), lambda b: (b, 0, 0)),
            pl.BlockSpec((_K * C_in, C_out), lambda b: (0, 0)),
            pl.BlockSpec((nblk_s, 2, C_out), lambda b: (0, 0, 0)),
            pl.BlockSpec((1, C_out), lambda b: (0, 0)),
            pl.BlockSpec((1, C_out), lambda b: (0, 0)),
        ],
        out_specs=pl.BlockSpec((_NB, L_out, C_out), lambda b: (b, 0, 0)),
        compiler_params=cparams,
    )(x16, w16, stats, g_row, bt_row)

    return y


def kernel(x, conv_weight, conv_bias, bn_gamma, bn_beta):
    return _forward(x, conv_weight, conv_bias, bn_gamma, bn_beta)
